# baseline (speedup 1.0000x reference)
import jax
import jax.numpy as jnp
from jax.experimental import pallas as pl
from jax.experimental.pallas import tpu as pltpu

_BM = 128


def _support_body(x_ref, w_ref, s_ref):
    w = w_ref[...]
    s0 = jnp.dot(x_ref[0], w, preferred_element_type=jnp.float32)
    s1 = jnp.dot(x_ref[1], w, preferred_element_type=jnp.float32)
    s_ref[...] = jnp.concatenate([s0, s1], axis=1)


def _spmm_body(adj_ref, s_ref, out_ref):
    a = adj_ref[...].astype(jnp.bfloat16)
    s = s_ref[...].astype(jnp.bfloat16)
    acc = jnp.dot(a, s, preferred_element_type=jnp.float32)
    out_ref[...] = jnp.maximum(acc, 0.0)


def kernel(input, adj, W):
    bs, n_agents, in_f = input.shape
    out_f = W.shape[1]

    support = pl.pallas_call(
        _support_body,
        out_shape=jax.ShapeDtypeStruct((n_agents, bs * out_f), jnp.float32),
    )(input, W)

    grid = (n_agents // _BM,)
    out = pl.pallas_call(
        _spmm_body,
        grid=grid,
        in_specs=[
            pl.BlockSpec((_BM, n_agents), lambda i: (i, 0)),
            pl.BlockSpec((n_agents, bs * out_f), lambda i: (0, 0)),
        ],
        out_specs=pl.BlockSpec((_BM, bs * out_f), lambda i: (i, 0)),
        out_shape=jax.ShapeDtypeStruct((n_agents, bs * out_f), jnp.float32),
        compiler_params=pltpu.CompilerParams(
            dimension_semantics=("parallel",),
        ),
    )(adj, support)

    out = out.reshape(n_agents, bs, out_f).transpose(1, 0, 2)
    return out.reshape(bs * n_agents, out_f)


# speedup vs baseline: 1.1871x; 1.1871x over previous
import jax
import jax.numpy as jnp
from jax.experimental import pallas as pl
from jax.experimental.pallas import tpu as pltpu

_BM = 256


def _support_body(x_ref, w_ref, s_ref):
    w = w_ref[...]
    s0 = jnp.dot(x_ref[0], w, preferred_element_type=jnp.float32)
    s1 = jnp.dot(x_ref[1], w, preferred_element_type=jnp.float32)
    s_ref[...] = jnp.concatenate([s0, s1], axis=1)


def _spmm_body(adj_ref, s_ref, out_ref):
    acc = jnp.dot(adj_ref[...], s_ref[...], preferred_element_type=jnp.float32)
    out_ref[...] = jnp.maximum(acc, 0.0)


def kernel(input, adj, W):
    bs, n_agents, in_f = input.shape
    out_f = W.shape[1]

    support = pl.pallas_call(
        _support_body,
        out_shape=jax.ShapeDtypeStruct((n_agents, bs * out_f), jnp.float32),
    )(input, W)

    grid = (n_agents // _BM,)
    out = pl.pallas_call(
        _spmm_body,
        grid=grid,
        in_specs=[
            pl.BlockSpec((_BM, n_agents), lambda i: (i, 0)),
            pl.BlockSpec((n_agents, bs * out_f), lambda i: (0, 0)),
        ],
        out_specs=pl.BlockSpec((_BM, bs * out_f), lambda i: (i, 0)),
        out_shape=jax.ShapeDtypeStruct((n_agents, bs * out_f), jnp.float32),
        compiler_params=pltpu.CompilerParams(
            dimension_semantics=("parallel",),
        ),
    )(adj, support)

    out = out.reshape(n_agents, bs, out_f).transpose(1, 0, 2)
    return out.reshape(bs * n_agents, out_f)


# speedup vs baseline: 1.2393x; 1.0440x over previous
import jax
import jax.numpy as jnp
from jax.experimental import pallas as pl
from jax.experimental.pallas import tpu as pltpu

_BM = 256


def _gcn_body(x_ref, w_ref, adj_ref, out_ref, s_vmem):
    @pl.when(pl.program_id(0) == 0)
    def _():
        w = w_ref[...]
        s0 = jnp.dot(x_ref[0], w, preferred_element_type=jnp.float32)
        s1 = jnp.dot(x_ref[1], w, preferred_element_type=jnp.float32)
        s_vmem[...] = jnp.concatenate([s0, s1], axis=1)

    acc = jnp.dot(adj_ref[...], s_vmem[...], preferred_element_type=jnp.float32)
    out_ref[...] = jnp.maximum(acc, 0.0)


def kernel(input, adj, W):
    bs, n_agents, in_f = input.shape
    out_f = W.shape[1]

    grid = (n_agents // _BM,)
    out = pl.pallas_call(
        _gcn_body,
        grid=grid,
        in_specs=[
            pl.BlockSpec((bs, n_agents, in_f), lambda i: (0, 0, 0)),
            pl.BlockSpec((in_f, out_f), lambda i: (0, 0)),
            pl.BlockSpec((_BM, n_agents), lambda i: (i, 0)),
        ],
        out_specs=pl.BlockSpec((_BM, bs * out_f), lambda i: (i, 0)),
        out_shape=jax.ShapeDtypeStruct((n_agents, bs * out_f), jnp.float32),
        scratch_shapes=[pltpu.VMEM((n_agents, bs * out_f), jnp.float32)],
        compiler_params=pltpu.CompilerParams(
            dimension_semantics=("arbitrary",),
        ),
    )(input, W, adj)

    out = out.reshape(n_agents, bs, out_f).transpose(1, 0, 2)
    return out.reshape(bs * n_agents, out_f)


# speedup vs baseline: 1.2405x; 1.0010x over previous
import jax
import jax.numpy as jnp
from jax.experimental import pallas as pl
from jax.experimental.pallas import tpu as pltpu

_BM = 512


def _gcn_body(x_ref, w_ref, adj_ref, out_ref, s_vmem):
    @pl.when(pl.program_id(0) == 0)
    def _():
        w = w_ref[...]
        s0 = jnp.dot(x_ref[0], w, preferred_element_type=jnp.float32)
        s1 = jnp.dot(x_ref[1], w, preferred_element_type=jnp.float32)
        s_vmem[...] = jnp.concatenate([s0, s1], axis=1)

    acc = jnp.dot(adj_ref[...], s_vmem[...], preferred_element_type=jnp.float32)
    out_ref[...] = jnp.maximum(acc, 0.0)


def kernel(input, adj, W):
    bs, n_agents, in_f = input.shape
    out_f = W.shape[1]

    grid = (n_agents // _BM,)
    out = pl.pallas_call(
        _gcn_body,
        grid=grid,
        in_specs=[
            pl.BlockSpec((bs, n_agents, in_f), lambda i: (0, 0, 0)),
            pl.BlockSpec((in_f, out_f), lambda i: (0, 0)),
            pl.BlockSpec((_BM, n_agents), lambda i: (i, 0)),
        ],
        out_specs=pl.BlockSpec((_BM, bs * out_f), lambda i: (i, 0)),
        out_shape=jax.ShapeDtypeStruct((n_agents, bs * out_f), jnp.float32),
        scratch_shapes=[pltpu.VMEM((n_agents, bs * out_f), jnp.float32)],
        compiler_params=pltpu.CompilerParams(
            dimension_semantics=("arbitrary",),
            vmem_limit_bytes=120 * 1024 * 1024,
        ),
    )(input, W, adj)

    out = out.reshape(n_agents, bs, out_f).transpose(1, 0, 2)
    return out.reshape(bs * n_agents, out_f)
